# Initial kernel scaffold; baseline (speedup 1.0000x reference)
import jax
import jax.numpy as jnp
from jax.experimental import pallas as pl


def kernel(msa, token_table, pos_table, row_table, gamma, beta):
    raise NotImplementedError("write your pallas kernel here")


# speedup vs baseline: 8.1491x; 8.1491x over previous
import functools

import jax
import jax.numpy as jnp
from jax.experimental import pallas as pl

EPS = 1e-5


def _tc_body(msa_ref, tok_ref, pos_ref, row_ref, gamma_ref, beta_ref, out_ref):
    rn, L = msa_ref.shape[1], msa_ref.shape[3]
    D = tok_ref.shape[1]
    t = msa_ref[0, :, 0, :]
    oh = (t[:, :, None] == jax.lax.broadcasted_iota(jnp.int32, (1, 1, 8), 2)
          ).astype(jnp.float32)
    tok = jnp.dot(oh.reshape(rn * L, 8), tok_ref[...],
                  preferred_element_type=jnp.float32)
    emb = (tok.reshape(rn, L, D)
           + pos_ref[...][None, :, :]
           + row_ref[...][:, None, :])
    mu = jnp.mean(emb, axis=-1, keepdims=True)
    var = jnp.mean((emb - mu) ** 2, axis=-1, keepdims=True)
    y = (emb - mu) * jax.lax.rsqrt(var + EPS)
    y = y * gamma_ref[...][None, None, :] + beta_ref[...][None, None, :]
    out_ref[0] = y


@functools.partial(jax.jit, static_argnames=("rn",))
def _tc_kernel(msa, token_table, pos_table, row_table, gamma, beta, rn=8):
    B, N, L = msa.shape
    V, D = token_table.shape
    msa4 = msa.astype(jnp.int32).reshape(B * N // rn, rn, 1, L)
    tok8 = jnp.zeros((8, D), jnp.float32).at[:V].set(token_table)
    grid = (B * N // rn,)
    out = pl.pallas_call(
        _tc_body,
        grid=grid,
        in_specs=[
            pl.BlockSpec((1, rn, 1, L), lambda i: (i, 0, 0, 0)),
            pl.BlockSpec((8, D), lambda i: (0, 0)),
            pl.BlockSpec((L, D), lambda i: (0, 0)),
            pl.BlockSpec((rn, D), lambda i: (i % (N // rn), 0)),
            pl.BlockSpec((D,), lambda i: (0,)),
            pl.BlockSpec((D,), lambda i: (0,)),
        ],
        out_specs=pl.BlockSpec((1, rn, L, D), lambda i: (i, 0, 0, 0)),
        out_shape=jax.ShapeDtypeStruct((B * N // rn, rn, L, D), jnp.float32),
    )(msa4, tok8, pos_table, row_table, gamma, beta)
    return out.reshape(B, N, L, D)


def kernel(msa, token_table, pos_table, row_table, gamma, beta):
    return _tc_kernel(msa, token_table, pos_table, row_table, gamma, beta)
